# baseline (speedup 1.0000x reference)
import functools

import jax
import jax.numpy as jnp
from jax import lax
from jax.experimental import pallas as pl
from jax.experimental.pallas import tpu as pltpu

_B = 4096
_D = 512
_N = 1000
_R = 1024
_G = _B // _R
_K = _D + 8
_H = _R // 2


def _tc_body(labels_ref, feats_ref, centers_ref, out_ref, cb_sc, f_aug):
    i = pl.program_id(0)

    @pl.when(i == 0)
    def _():
        C = centers_ref[...]
        cn = jnp.sum(C * C, axis=1, keepdims=True)
        cb_sc[...] = jnp.concatenate(
            [-2.0 * C, cn, jnp.zeros((_N, 7), jnp.float32)],
            axis=1).astype(jnp.bfloat16)
        f_aug[:, _D:] = jnp.concatenate(
            [jnp.ones((_R, 1), jnp.bfloat16),
             jnp.zeros((_R, 7), jnp.bfloat16)], axis=1)

    F = feats_ref[...]
    f2 = jnp.sum(F * F)
    f_aug[:, :_D] = F.astype(jnp.bfloat16)
    lab = labels_ref[...]
    row = lax.broadcasted_iota(jnp.int32, (_N, 1), 0)

    sel = jnp.float32(0.0)
    for h in range(2):
        Pt = lax.dot_general(
            cb_sc[...], f_aug[pl.ds(h * _H, _H), :],
            (((1,), (1,)), ((), ())), preferred_element_type=jnp.float32)
        mask = row == lab[:, h * _H:(h + 1) * _H]
        sel += jnp.sum(jnp.where(mask, Pt, 0.0))

    contrib = jnp.reshape(sel + f2, (1, 1))

    @pl.when(i == 0)
    def _():
        out_ref[...] = contrib

    @pl.when(i > 0)
    def _():
        out_ref[...] += contrib


def kernel(feats, labels, centers):
    lab2 = labels.astype(jnp.int32).reshape(1, _B)
    out = pl.pallas_call(
        _tc_body,
        grid=(_G,),
        in_specs=[
            pl.BlockSpec((1, _R), lambda i: (0, i)),
            pl.BlockSpec((_R, _D), lambda i: (i, 0)),
            pl.BlockSpec((_N, _D), lambda i: (0, 0)),
        ],
        out_specs=pl.BlockSpec((1, 1), lambda i: (0, 0)),
        out_shape=jax.ShapeDtypeStruct((1, 1), jnp.float32),
        scratch_shapes=[
            pltpu.VMEM((_N, _K), jnp.bfloat16),
            pltpu.VMEM((_R, _K), jnp.bfloat16),
        ],
    )(lab2, feats, centers)
    return out[0, 0] / jnp.float32(_B * _D)


# speedup vs baseline: 1.1982x; 1.1982x over previous
import functools

import jax
import jax.numpy as jnp
from jax import lax
from jax.experimental import pallas as pl
from jax.experimental.pallas import tpu as pltpu

_B = 4096
_D = 512
_N = 1000
_R = 1024
_G = _B // _R


def _tc_body(labels_ref, feats_ref, centers_ref, out_ref, cb_sc, cn_col,
             acc_b, acc_d):
    i = pl.program_id(0)

    @pl.when(i == 0)
    def _():
        C = centers_ref[...]
        cb_sc[...] = (-2.0 * C).astype(jnp.bfloat16)
        cn_col[...] = jnp.sum(C * C, axis=1, keepdims=True)
        acc_b[...] = jnp.zeros((1, _R), jnp.float32)
        acc_d[...] = jnp.zeros((1, _D), jnp.float32)

    F = feats_ref[...]
    Pt = lax.dot_general(
        cb_sc[...], F.astype(jnp.bfloat16),
        (((1,), (1,)), ((), ())), preferred_element_type=jnp.float32)
    lab = labels_ref[...]
    row = lax.broadcasted_iota(jnp.int32, (_N, 1), 0)
    mask = row == lab
    psel = jnp.where(mask, cn_col[...] + Pt, 0.0)

    ones_n = jnp.ones((1, _N), jnp.float32)
    acc_b[...] += lax.dot_general(
        ones_n, psel, (((1,), (0,)), ((), ())),
        preferred_element_type=jnp.float32)

    ff = F * F
    ones_r = jnp.ones((1, _R), jnp.float32)
    acc_d[...] += lax.dot_general(
        ones_r, ff, (((1,), (0,)), ((), ())),
        preferred_element_type=jnp.float32)

    @pl.when(i == _G - 1)
    def _():
        out_ref[...] = jnp.reshape(
            jnp.sum(acc_b[...]) + jnp.sum(acc_d[...]), (1, 1))


def kernel(feats, labels, centers):
    lab2 = labels.astype(jnp.int32).reshape(1, _B)
    out = pl.pallas_call(
        _tc_body,
        grid=(_G,),
        in_specs=[
            pl.BlockSpec((1, _R), lambda i: (0, i)),
            pl.BlockSpec((_R, _D), lambda i: (i, 0)),
            pl.BlockSpec((_N, _D), lambda i: (0, 0)),
        ],
        out_specs=pl.BlockSpec((1, 1), lambda i: (0, 0)),
        out_shape=jax.ShapeDtypeStruct((1, 1), jnp.float32),
        scratch_shapes=[
            pltpu.VMEM((_N, _D), jnp.bfloat16),
            pltpu.VMEM((_N, 1), jnp.float32),
            pltpu.VMEM((1, _R), jnp.float32),
            pltpu.VMEM((1, _D), jnp.float32),
        ],
    )(lab2, feats, centers)
    return out[0, 0] / jnp.float32(_B * _D)


# speedup vs baseline: 1.2196x; 1.0179x over previous
import functools

import jax
import jax.numpy as jnp
from jax import lax
from jax.experimental import pallas as pl
from jax.experimental.pallas import tpu as pltpu

_B = 4096
_D = 512
_N = 1000
_R = 2048
_G = _B // _R


def _tc_body(labels_ref, feats_ref, centers_ref, out_ref, cb_sc, cn_col,
             acc_b, acc_d):
    i = pl.program_id(0)

    @pl.when(i == 0)
    def _():
        C = centers_ref[...]
        cb_sc[...] = (-2.0 * C).astype(jnp.bfloat16)
        cn_col[...] = jnp.sum(C * C, axis=1, keepdims=True)
        acc_b[...] = jnp.zeros((1, _R), jnp.float32)
        acc_d[...] = jnp.zeros((1, _D), jnp.float32)

    F = feats_ref[...]
    Pt = lax.dot_general(
        cb_sc[...], F.astype(jnp.bfloat16),
        (((1,), (1,)), ((), ())), preferred_element_type=jnp.float32)
    lab = labels_ref[...]
    row = lax.broadcasted_iota(jnp.int32, (_N, 1), 0)
    mask = row == lab
    psel = jnp.where(mask, cn_col[...] + Pt, 0.0)

    ones_n = jnp.ones((1, _N), jnp.float32)
    acc_b[...] += lax.dot_general(
        ones_n, psel, (((1,), (0,)), ((), ())),
        preferred_element_type=jnp.float32)

    ff = F * F
    ones_r = jnp.ones((1, _R), jnp.float32)
    acc_d[...] += lax.dot_general(
        ones_r, ff, (((1,), (0,)), ((), ())),
        preferred_element_type=jnp.float32)

    @pl.when(i == _G - 1)
    def _():
        out_ref[...] = jnp.reshape(
            jnp.sum(acc_b[...]) + jnp.sum(acc_d[...]), (1, 1))


def kernel(feats, labels, centers):
    lab2 = labels.astype(jnp.int32).reshape(1, _B)
    out = pl.pallas_call(
        _tc_body,
        grid=(_G,),
        in_specs=[
            pl.BlockSpec((1, _R), lambda i: (0, i)),
            pl.BlockSpec((_R, _D), lambda i: (i, 0)),
            pl.BlockSpec((_N, _D), lambda i: (0, 0)),
        ],
        out_specs=pl.BlockSpec((1, 1), lambda i: (0, 0)),
        out_shape=jax.ShapeDtypeStruct((1, 1), jnp.float32),
        scratch_shapes=[
            pltpu.VMEM((_N, _D), jnp.bfloat16),
            pltpu.VMEM((_N, 1), jnp.float32),
            pltpu.VMEM((1, _R), jnp.float32),
            pltpu.VMEM((1, _D), jnp.float32),
        ],
    )(lab2, feats, centers)
    return out[0, 0] / jnp.float32(_B * _D)
